# Initial kernel scaffold; baseline (speedup 1.0000x reference)
import jax
import jax.numpy as jnp
from jax.experimental import pallas as pl


def kernel(x, edge_index, W1l, W1r, b1, W2l, W2r, b2):
    raise NotImplementedError("write your pallas kernel here")


# speedup vs baseline: 6.2754x; 6.2754x over previous
import functools

import jax
import jax.numpy as jnp
from jax import lax
from jax.experimental import pallas as pl
from jax.experimental.pallas import tpu as pltpu
from jax.experimental.pallas import tpu_sc as plsc

_N = 10000
_E = 320000
_F = 128
_HID = 128
_CLS = 32

_NC = 2
_NS = 16
_NW = _NC * _NS
_CHUNK = 128
_NCHUNK = -(-_E // (_NW * _CHUNK))
_EPW = _NCHUNK * _CHUNK
_EPAD = _EPW * _NW
_NPAD = 10112
_RPT = _NPAD // _NS

_BLK = 1000
_GRID = _N // _BLK


def _deg_body(dstw, zd, oh, out_d, dst_v, oh_v, acc_d):
    cid = lax.axis_index("c")
    sid = lax.axis_index("s")
    wid = cid * _NS + sid
    r0 = sid * _RPT
    pltpu.sync_copy(zd.at[pl.ds(r0, _RPT)], acc_d.at[pl.ds(r0, _RPT)])
    pltpu.sync_copy(dstw.at[wid], dst_v)
    pltpu.sync_copy(oh, oh_v)
    plsc.subcore_barrier()

    def body(j, carry):
        pltpu.sync_copy(oh_v, acc_d.at[dst_v.at[j]], add=True)
        return carry

    lax.fori_loop(0, _NCHUNK, body, 0)
    plsc.subcore_barrier()
    pltpu.sync_copy(acc_d.at[pl.ds(r0, _RPT)], out_d.at[cid, pl.ds(r0, _RPT)])


def _agg1_body(feat, srcw, dstw, zf, out_f,
               src_v, dst_v, rows_v, acc_f):
    cid = lax.axis_index("c")
    sid = lax.axis_index("s")
    wid = cid * _NS + sid
    r0 = sid * _RPT
    pltpu.sync_copy(zf.at[pl.ds(r0, _RPT)], acc_f.at[pl.ds(r0, _RPT)])
    pltpu.sync_copy(srcw.at[wid], src_v)
    pltpu.sync_copy(dstw.at[wid], dst_v)
    plsc.subcore_barrier()

    def body(j, carry):
        pltpu.sync_copy(feat.at[src_v.at[j]], rows_v)
        pltpu.sync_copy(rows_v, acc_f.at[dst_v.at[j]], add=True)
        return carry

    lax.fori_loop(0, _NCHUNK, body, 0)
    plsc.subcore_barrier()
    pltpu.sync_copy(acc_f.at[pl.ds(r0, _RPT)], out_f.at[cid, pl.ds(r0, _RPT)])


def _agg2_body(feat, srcw, dstw, z2, out_f,
               src_v, dst_v, rows_v, acc_f):
    cid = lax.axis_index("c")
    sid = lax.axis_index("s")
    wid = cid * _NS + sid
    r0 = sid * _RPT
    pltpu.sync_copy(z2.at[pl.ds(r0, _RPT)], acc_f.at[pl.ds(r0, _RPT)])
    pltpu.sync_copy(srcw.at[wid], src_v)
    pltpu.sync_copy(dstw.at[wid], dst_v)
    plsc.subcore_barrier()

    def body(j, carry):
        pltpu.sync_copy(feat.at[src_v.at[j]], rows_v)
        pltpu.sync_copy(rows_v, acc_f.at[dst_v.at[j]], add=True)
        return carry

    lax.fori_loop(0, _NCHUNK, body, 0)
    plsc.subcore_barrier()
    pltpu.sync_copy(acc_f.at[pl.ds(r0, _RPT)], out_f.at[cid, pl.ds(r0, _RPT)])


def _make_deg():
    return functools.partial(
        pl.kernel,
        out_type=jax.ShapeDtypeStruct((_NC, _NPAD, 16), jnp.float32),
        mesh=plsc.VectorSubcoreMesh(core_axis_name="c", subcore_axis_name="s"),
        compiler_params=pltpu.CompilerParams(use_tc_tiling_on_sc=False),
        scratch_types=[
            pltpu.VMEM((_NCHUNK, _CHUNK), jnp.int32),
            pltpu.VMEM((_CHUNK, 16), jnp.float32),
            pltpu.VMEM_SHARED((_NPAD, 16), jnp.float32),
        ],
    )(_deg_body)


def _make_agg1():
    return functools.partial(
        pl.kernel,
        out_type=jax.ShapeDtypeStruct((_NC, _NPAD, _F), jnp.float32),
        mesh=plsc.VectorSubcoreMesh(core_axis_name="c", subcore_axis_name="s"),
        scratch_types=[
            pltpu.VMEM((_NCHUNK, _CHUNK), jnp.int32),
            pltpu.VMEM((_NCHUNK, _CHUNK), jnp.int32),
            pltpu.VMEM((_CHUNK, _F), jnp.float32),
            pltpu.VMEM_SHARED((_NPAD, _F), jnp.float32),
        ],
    )(_agg1_body)


def _make_agg2():
    return functools.partial(
        pl.kernel,
        out_type=jax.ShapeDtypeStruct((_NC, _NPAD, _CLS), jnp.float32),
        mesh=plsc.VectorSubcoreMesh(core_axis_name="c", subcore_axis_name="s"),
        compiler_params=pltpu.CompilerParams(use_tc_tiling_on_sc=False),
        scratch_types=[
            pltpu.VMEM((_NCHUNK, _CHUNK), jnp.int32),
            pltpu.VMEM((_NCHUNK, _CHUNK), jnp.int32),
            pltpu.VMEM((_CHUNK, _CLS), jnp.float32),
            pltpu.VMEM_SHARED((_NPAD, _CLS), jnp.float32),
        ],
    )(_agg2_body)


def _sc_degree(dstw):
    zd = jnp.zeros((_NPAD, 16), jnp.float32)
    oh = jnp.zeros((_CHUNK, 16), jnp.float32).at[:, 0].set(1.0)
    return _make_deg()(dstw, zd, oh)


def _sc_aggregate1(p1, srcw, dstw):
    zf = jnp.zeros((_NPAD, _F), jnp.float32)
    return _make_agg1()(p1, srcw, dstw, zf)


def _sc_aggregate2(p2, srcw, dstw):
    z2 = jnp.zeros((_NPAD, _CLS), jnp.float32)
    return _make_agg2()(p2, srcw, dstw, z2)


def _prep_body(x_ref, w1l_ref, w1r_ref, b1_ref, p1_ref, r1_ref):
    xb = x_ref[...]
    p1_ref[...] = jnp.dot(xb, w1l_ref[...], preferred_element_type=jnp.float32)
    r1_ref[...] = (jnp.dot(xb, w1r_ref[...], preferred_element_type=jnp.float32)
                   + b1_ref[...])


def _mid_body(pf_ref, pd_ref, r1_ref, w2l_ref, w2r_ref, b2_ref,
              p2_ref, r2_ref, inv_ref):
    agg = pf_ref[0] + pf_ref[1]
    degc = pd_ref[0] + pd_ref[1]
    invd = 1.0 / jnp.maximum(degc[:, 0:1], 1.0)
    h = jnp.maximum(agg * invd + r1_ref[...], 0.0)
    p2_ref[...] = jnp.dot(h, w2l_ref[...], preferred_element_type=jnp.float32)
    r2_ref[...] = (jnp.dot(h, w2r_ref[...], preferred_element_type=jnp.float32)
                   + b2_ref[...])
    inv_ref[...] = jnp.broadcast_to(invd, (invd.shape[0], _CLS))


def _final_body(pf_ref, r2_ref, inv_ref, o_ref):
    y = (pf_ref[0] + pf_ref[1]) * inv_ref[...] + r2_ref[...]
    m = jnp.max(y, axis=1, keepdims=True)
    lse = jnp.log(jnp.sum(jnp.exp(y - m), axis=1, keepdims=True)) + m
    o_ref[...] = y - lse


def _tc_prep(x, w1lT, w1rT, b1):
    return pl.pallas_call(
        _prep_body,
        grid=(_GRID,),
        in_specs=[
            pl.BlockSpec((_BLK, _F), lambda i: (i, 0)),
            pl.BlockSpec((_F, _HID), lambda i: (0, 0)),
            pl.BlockSpec((_F, _HID), lambda i: (0, 0)),
            pl.BlockSpec((1, _HID), lambda i: (0, 0)),
        ],
        out_specs=[pl.BlockSpec((_BLK, _HID), lambda i: (i, 0))] * 2,
        out_shape=[jax.ShapeDtypeStruct((_N, _HID), jnp.float32)] * 2,
    )(x, w1lT, w1rT, b1)


def _tc_mid(pf, pd, r1, w2lT, w2rT, b2):
    return pl.pallas_call(
        _mid_body,
        grid=(_GRID,),
        in_specs=[
            pl.BlockSpec((_NC, _BLK, _F), lambda i: (0, i, 0)),
            pl.BlockSpec((_NC, _BLK, 16), lambda i: (0, i, 0)),
            pl.BlockSpec((_BLK, _HID), lambda i: (i, 0)),
            pl.BlockSpec((_HID, _CLS), lambda i: (0, 0)),
            pl.BlockSpec((_HID, _CLS), lambda i: (0, 0)),
            pl.BlockSpec((1, _CLS), lambda i: (0, 0)),
        ],
        out_specs=[pl.BlockSpec((_BLK, _CLS), lambda i: (i, 0))] * 3,
        out_shape=[jax.ShapeDtypeStruct((_N, _CLS), jnp.float32)] * 3,
    )(pf, pd, r1, w2lT, w2rT, b2)


def _tc_final(pf2, r2, inv):
    return pl.pallas_call(
        _final_body,
        grid=(_GRID,),
        in_specs=[
            pl.BlockSpec((_NC, _BLK, _CLS), lambda i: (0, i, 0)),
            pl.BlockSpec((_BLK, _CLS), lambda i: (i, 0)),
            pl.BlockSpec((_BLK, _CLS), lambda i: (i, 0)),
        ],
        out_specs=pl.BlockSpec((_BLK, _CLS), lambda i: (i, 0)),
        out_shape=jax.ShapeDtypeStruct((_N, _CLS), jnp.float32),
    )(pf2, r2, inv)


def kernel(x, edge_index, W1l, W1r, b1, W2l, W2r, b2):
    src = edge_index[0]
    dst = edge_index[1]
    pad = _EPAD - _E
    srcw = jnp.concatenate(
        [src, jnp.zeros((pad,), jnp.int32)]).reshape(_NW, _NCHUNK, _CHUNK)
    dstw = jnp.concatenate(
        [dst, jnp.full((pad,), _N, jnp.int32)]).reshape(_NW, _NCHUNK, _CHUNK)

    pd = _sc_degree(dstw)
    p1, r1 = _tc_prep(x, W1l.T, W1r.T, b1.reshape(1, _HID))
    pf = _sc_aggregate1(p1, srcw, dstw)
    p2, r2, inv = _tc_mid(pf, pd, r1, W2l.T, W2r.T, b2.reshape(1, _CLS))
    pf2 = _sc_aggregate2(p2, srcw, dstw)
    return _tc_final(pf2, r2, inv)
